# baseline (speedup 1.0000x reference)
import jax
import jax.numpy as jnp
from jax.experimental import pallas as pl

_B, _H, _W = 8, 512, 512
_N = _H * _W
_BB = 4
_STEPS = _B // _BB
_RC = 32
_NSTRIPS = _H // _RC


def _batch_partial(p_ref, t_ref, bb):
    def body(j, carry):
        av, atv = carry
        ps = p_ref[bb, pl.ds(j * _RC, _RC), :]
        ts = t_ref[bb, pl.ds(j * _RC, _RC), :]
        selv = jnp.where(ts == 1.0, ps, 1.0 - ps)
        v = jnp.maximum(jnp.log(selv), -100.0)
        return av + v, atv + ts * v

    zero = jnp.zeros((_RC, _W), jnp.float32)
    av, atv = jax.lax.fori_loop(0, _NSTRIPS, body, (zero, zero))
    totalv = jnp.sum(av)
    s1 = jnp.sum(atv)
    c1 = jnp.sum(t_ref[bb])
    s0 = totalv - s1
    return c1 * s1 + (_N - c1) * s0


def _bce_kernel(p_ref, t_ref, out_ref):
    b = pl.program_id(0)
    acc = _batch_partial(p_ref, t_ref, 0)
    for bb in range(1, _BB):
        acc += _batch_partial(p_ref, t_ref, bb)
    contrib = acc * (-1.0 / (_N * float(_N) * _B))

    @pl.when(b == 0)
    def _init():
        out_ref[:, :] = jnp.zeros((1, 1), jnp.float32)

    out_ref[:, :] += jnp.full((1, 1), contrib)


def kernel(predict, target):
    out = pl.pallas_call(
        _bce_kernel,
        grid=(_STEPS,),
        in_specs=[
            pl.BlockSpec((_BB, _H, _W), lambda b: (b, 0, 0)),
            pl.BlockSpec((_BB, _H, _W), lambda b: (b, 0, 0)),
        ],
        out_specs=pl.BlockSpec((1, 1), lambda b: (0, 0)),
        out_shape=jax.ShapeDtypeStruct((1, 1), jnp.float32),
    )(predict, target)
    return out[0, 0]


# speedup vs baseline: 1.3000x; 1.3000x over previous
import jax
import jax.numpy as jnp
from jax.experimental import pallas as pl

_B, _H, _W = 8, 512, 512
_N = _H * _W
_BB = 4
_STEPS = _B // _BB


def _bce_kernel(p_ref, t_ref, out_ref):
    b = pl.program_id(0)
    p = p_ref[...]
    t = t_ref[...]
    sel = jnp.abs(p + (t - 1.0))
    v = jnp.maximum(jnp.log(sel), -100.0)
    totalv = jnp.sum(v, axis=(1, 2))
    s1v = jnp.sum(t * v, axis=(1, 2))
    c1v = jnp.sum(t, axis=(1, 2))
    s0v = totalv - s1v
    contrib = jnp.sum(c1v * s1v + (_N - c1v) * s0v) * (
        -1.0 / (_N * float(_N) * _B)
    )

    @pl.when(b == 0)
    def _init():
        out_ref[:, :] = jnp.zeros((1, 1), jnp.float32)

    out_ref[:, :] += jnp.full((1, 1), contrib)


def kernel(predict, target):
    out = pl.pallas_call(
        _bce_kernel,
        grid=(_STEPS,),
        in_specs=[
            pl.BlockSpec((_BB, _H, _W), lambda b: (b, 0, 0)),
            pl.BlockSpec((_BB, _H, _W), lambda b: (b, 0, 0)),
        ],
        out_specs=pl.BlockSpec((1, 1), lambda b: (0, 0)),
        out_shape=jax.ShapeDtypeStruct((1, 1), jnp.float32),
    )(predict, target)
    return out[0, 0]


# speedup vs baseline: 1.3324x; 1.0249x over previous
import jax
import jax.numpy as jnp
from jax.experimental import pallas as pl

_B, _H, _W = 8, 512, 512
_N = _H * _W
_BB = 4
_STEPS = _B // _BB


def _bce_kernel(p_ref, t_ref, out_ref):
    b = pl.program_id(0)
    p = p_ref[...]
    t = t_ref[...]
    sel = jnp.where(t == 1.0, p, 1.0 - p)
    v = jnp.maximum(jnp.log(sel), -100.0)
    totalv = jnp.sum(v, axis=(1, 2))
    s1v = jnp.sum(t * v, axis=(1, 2))
    c1v = jnp.sum(t, axis=(1, 2))
    s0v = totalv - s1v
    contrib = jnp.sum(c1v * s1v + (_N - c1v) * s0v) * (
        -1.0 / (_N * float(_N) * _B)
    )

    @pl.when(b == 0)
    def _init():
        out_ref[:, :] = jnp.zeros((1, 1), jnp.float32)

    out_ref[:, :] += jnp.full((1, 1), contrib)


def kernel(predict, target):
    out = pl.pallas_call(
        _bce_kernel,
        grid=(_STEPS,),
        in_specs=[
            pl.BlockSpec((_BB, _H, _W), lambda b: (b, 0, 0)),
            pl.BlockSpec((_BB, _H, _W), lambda b: (b, 0, 0)),
        ],
        out_specs=pl.BlockSpec((1, 1), lambda b: (0, 0)),
        out_shape=jax.ShapeDtypeStruct((1, 1), jnp.float32),
    )(predict, target)
    return out[0, 0]
